# baseline (speedup 1.0000x reference)
import jax
import jax.numpy as jnp
from jax.experimental import pallas as pl
from jax.experimental.pallas import tpu as pltpu

_COL = 777
_B = 16384


_NQ = 8
_W = _B // _NQ


def _copy_body(tvT_ref, out_ref, sem):
    copies = [
        pltpu.make_async_copy(
            tvT_ref.at[_COL, pl.ds(i * _W, _W)],
            out_ref.at[pl.ds(i * _W, _W)],
            sem,
        )
        for i in range(_NQ)
    ]
    for c in copies:
        c.start()
    for c in copies:
        c.wait()


def kernel(truth_values):
    tvT = truth_values.T
    flat = pl.pallas_call(
        _copy_body,
        in_specs=[pl.BlockSpec(memory_space=pl.ANY)],
        out_specs=pl.BlockSpec(memory_space=pl.ANY),
        out_shape=jax.ShapeDtypeStruct((_B,), jnp.float32),
        scratch_shapes=[pltpu.SemaphoreType.DMA],
    )(tvT)
    return flat.reshape(_B, 1)


# speedup vs baseline: 1.7438x; 1.7438x over previous
import jax
import jax.numpy as jnp
from jax.experimental import pallas as pl
from jax.experimental.pallas import tpu as pltpu

_COL = 777
_B = 16384
_ROW_BASE = (_COL // 8) * 8
_SUBLANE = _COL % 8


def _extract_body(tvT_hbm, out_hbm, buf, obuf, sem):
    cp_in = pltpu.make_async_copy(tvT_hbm.at[pl.ds(_ROW_BASE, 8), :], buf, sem)
    cp_in.start()
    cp_in.wait()
    obuf[...] = buf[_SUBLANE, :].reshape(_B // 128, 128)
    cp_out = pltpu.make_async_copy(obuf, out_hbm, sem)
    cp_out.start()
    cp_out.wait()


def kernel(truth_values):
    tvT = truth_values.T
    packed = pl.pallas_call(
        _extract_body,
        in_specs=[pl.BlockSpec(memory_space=pl.ANY)],
        out_specs=pl.BlockSpec(memory_space=pl.ANY),
        out_shape=jax.ShapeDtypeStruct((_B // 128, 128), jnp.float32),
        scratch_shapes=[
            pltpu.VMEM((8, _B), jnp.float32),
            pltpu.VMEM((_B // 128, 128), jnp.float32),
            pltpu.SemaphoreType.DMA,
        ],
    )(tvT)
    return packed.reshape(_B, 1)


# speedup vs baseline: 1.7683x; 1.0141x over previous
import jax
import jax.numpy as jnp
from jax.experimental import pallas as pl

_COL = 777
_B = 16384
_ROW_TILE = _COL // 8
_SUBLANE = _COL % 8
_COLS_PER_STEP = 16384


def _extract_body(tvT_ref, out_ref):
    strip = tvT_ref[_SUBLANE, :]
    out_ref[...] = strip.reshape(_COLS_PER_STEP // 128, 128)


def kernel(truth_values):
    tvT = truth_values.T
    grid = (_B // _COLS_PER_STEP,)
    packed = pl.pallas_call(
        _extract_body,
        grid=grid,
        in_specs=[
            pl.BlockSpec((8, _COLS_PER_STEP), lambda i: (_ROW_TILE, i)),
        ],
        out_specs=pl.BlockSpec((_COLS_PER_STEP // 128, 128), lambda i: (i, 0)),
        out_shape=jax.ShapeDtypeStruct((_B // 128, 128), jnp.float32),
    )(tvT)
    return packed.reshape(_B, 1)
